# baseline (speedup 1.0000x reference)
import jax
import jax.numpy as jnp
from jax.experimental import pallas as pl
from jax.experimental.pallas import tpu as pltpu

NFEAT = 128
NHID = 32
ALPHA = 0.2
N_NODES = 10000
N_CLUSTERS = 10

BR = 200
NI = N_NODES // BR
NBUF = 5
NJ = NI // NBUF


def _soft_assign(o, mu):
    cols = []
    for c in range(N_CLUSTERS):
        d = o - mu[c:c + 1, :]
        cols.append(jnp.sum(d * d, axis=1, keepdims=True))
    dist2 = jnp.concatenate(cols, axis=1)
    qv = 1.0 / (1.0 + dist2 / ALPHA + 1e-8)
    p = jnp.exp((ALPHA + 1.0) * jnp.log(qv))
    return p / jnp.sum(p, axis=1, keepdims=True)


def _gcdec_kernel(x_hbm, adj_hbm, w_ref, b_ref, mu_ref, out_hbm, q_hbm,
                  x_vmem, sup_ref, adj_buf, out_buf, q_buf,
                  x_sem, adj_sem, out_sem, q_sem):
    def adj_copy(block, slot):
        return pltpu.make_async_copy(
            adj_hbm.at[pl.ds(block * BR, BR), :], adj_buf.at[slot],
            adj_sem.at[slot])

    def out_copies(block, slot):
        rows = pl.ds(block * BR, BR)
        return (
            pltpu.make_async_copy(out_buf.at[slot], out_hbm.at[rows, :],
                                  out_sem.at[slot]),
            pltpu.make_async_copy(q_buf.at[slot], q_hbm.at[rows, :],
                                  q_sem.at[slot]),
        )

    cp_x = pltpu.make_async_copy(x_hbm, x_vmem, x_sem)
    cp_x.start()
    for p in range(NBUF):
        adj_copy(p, p).start()
    cp_x.wait()
    sup_ref[...] = jnp.dot(x_vmem[...], w_ref[...],
                           preferred_element_type=jnp.float32)

    def body(j, carry):
        for p in range(NBUF):
            block = j * NBUF + p
            adj_copy(block, p).wait()
            o = jnp.dot(adj_buf[p], sup_ref[...],
                        preferred_element_type=jnp.float32) + b_ref[...]
            q = _soft_assign(o, mu_ref[...])

            @pl.when(j >= 1)
            def _():
                co, cq = out_copies(block - NBUF, p)
                co.wait()
                cq.wait()

            out_buf[p] = o
            q_buf[p] = q
            co, cq = out_copies(block, p)
            co.start()
            cq.start()

            @pl.when(j + 1 < NJ)
            def _():
                adj_copy(block + NBUF, p).start()
        return carry

    jax.lax.fori_loop(0, NJ, body, 0, unroll=False)
    for p in range(NBUF):
        co, cq = out_copies(NI - NBUF + p, p)
        co.wait()
        cq.wait()


@jax.jit
def kernel(x, adj, W, b, mu):
    b2 = b.reshape(1, NHID)
    out, q = pl.pallas_call(
        _gcdec_kernel,
        in_specs=[
            pl.BlockSpec(memory_space=pl.ANY),
            pl.BlockSpec(memory_space=pl.ANY),
            pl.BlockSpec((NFEAT, NHID), lambda: (0, 0)),
            pl.BlockSpec((1, NHID), lambda: (0, 0)),
            pl.BlockSpec((N_CLUSTERS, NHID), lambda: (0, 0)),
        ],
        out_specs=[
            pl.BlockSpec(memory_space=pl.ANY),
            pl.BlockSpec(memory_space=pl.ANY),
        ],
        out_shape=[
            jax.ShapeDtypeStruct((N_NODES, NHID), jnp.float32),
            jax.ShapeDtypeStruct((N_NODES, N_CLUSTERS), jnp.float32),
        ],
        scratch_shapes=[
            pltpu.VMEM((N_NODES, NFEAT), jnp.float32),
            pltpu.VMEM((N_NODES, NHID), jnp.float32),
            pltpu.VMEM((NBUF, BR, N_NODES), jnp.float32),
            pltpu.VMEM((NBUF, BR, NHID), jnp.float32),
            pltpu.VMEM((NBUF, BR, N_CLUSTERS), jnp.float32),
            pltpu.SemaphoreType.DMA,
            pltpu.SemaphoreType.DMA((NBUF,)),
            pltpu.SemaphoreType.DMA((NBUF,)),
            pltpu.SemaphoreType.DMA((NBUF,)),
        ],
    )(x, adj, W, b2, mu)
    return (out, q)


# speedup vs baseline: 1.0051x; 1.0051x over previous
import jax
import jax.numpy as jnp
from jax.experimental import pallas as pl
from jax.experimental.pallas import tpu as pltpu

NFEAT = 128
NHID = 32
ALPHA = 0.2
N_NODES = 10000
N_CLUSTERS = 10

BR = 200
NI = N_NODES // BR
NBUF = 5
NJ = NI // NBUF


def _soft_assign(o, mu):
    cols = []
    for c in range(N_CLUSTERS):
        d = o - mu[c:c + 1, :]
        cols.append(jnp.sum(d * d, axis=1, keepdims=True))
    dist2 = jnp.concatenate(cols, axis=1)
    qv = 1.0 / (1.0 + dist2 / ALPHA + 1e-8)
    p = jnp.exp((ALPHA + 1.0) * jnp.log(qv))
    return p / jnp.sum(p, axis=1, keepdims=True)


def _gcdec_kernel(x_hbm, adj_hbm, w_ref, b_ref, mu_ref, out_hbm, q_hbm,
                  *refs):
    adj_bufs = refs[0:NBUF]
    out_bufs = refs[NBUF:2 * NBUF]
    q_bufs = refs[2 * NBUF:3 * NBUF]
    x_vmem, sup_ref, x_sem, adj_sem, out_sem, q_sem = refs[3 * NBUF:]

    def adj_copy(block, slot):
        return pltpu.make_async_copy(
            adj_hbm.at[pl.ds(block * BR, BR), :], adj_bufs[slot],
            adj_sem.at[slot])

    def out_copies(block, slot):
        rows = pl.ds(block * BR, BR)
        return (
            pltpu.make_async_copy(out_bufs[slot], out_hbm.at[rows, :],
                                  out_sem.at[slot]),
            pltpu.make_async_copy(q_bufs[slot], q_hbm.at[rows, :],
                                  q_sem.at[slot]),
        )

    cp_x = pltpu.make_async_copy(x_hbm, x_vmem, x_sem)
    cp_x.start()
    for p in range(NBUF):
        adj_copy(p, p).start()
    cp_x.wait()
    sup_ref[...] = jnp.dot(x_vmem[...], w_ref[...],
                           preferred_element_type=jnp.float32)

    def body(j, carry):
        for p in range(NBUF):
            block = j * NBUF + p
            adj_copy(block, p).wait()
            o = jnp.dot(adj_bufs[p][...], sup_ref[...],
                        preferred_element_type=jnp.float32) + b_ref[...]
            q = _soft_assign(o, mu_ref[...])

            @pl.when(j >= 1)
            def _():
                co, cq = out_copies(block - NBUF, p)
                co.wait()
                cq.wait()

            out_bufs[p][...] = o
            q_bufs[p][...] = q
            co, cq = out_copies(block, p)
            co.start()
            cq.start()

            @pl.when(j + 1 < NJ)
            def _():
                adj_copy(block + NBUF, p).start()
        return carry

    jax.lax.fori_loop(0, NJ, body, 0, unroll=False)
    for p in range(NBUF):
        co, cq = out_copies(NI - NBUF + p, p)
        co.wait()
        cq.wait()


@jax.jit
def kernel(x, adj, W, b, mu):
    b2 = b.reshape(1, NHID)
    out, q = pl.pallas_call(
        _gcdec_kernel,
        in_specs=[
            pl.BlockSpec(memory_space=pl.ANY),
            pl.BlockSpec(memory_space=pl.ANY),
            pl.BlockSpec((NFEAT, NHID), lambda: (0, 0)),
            pl.BlockSpec((1, NHID), lambda: (0, 0)),
            pl.BlockSpec((N_CLUSTERS, NHID), lambda: (0, 0)),
        ],
        out_specs=[
            pl.BlockSpec(memory_space=pl.ANY),
            pl.BlockSpec(memory_space=pl.ANY),
        ],
        out_shape=[
            jax.ShapeDtypeStruct((N_NODES, NHID), jnp.float32),
            jax.ShapeDtypeStruct((N_NODES, N_CLUSTERS), jnp.float32),
        ],
        scratch_shapes=(
            [pltpu.VMEM((BR, N_NODES), jnp.float32) for _ in range(NBUF)]
            + [pltpu.VMEM((BR, NHID), jnp.float32) for _ in range(NBUF)]
            + [pltpu.VMEM((BR, N_CLUSTERS), jnp.float32) for _ in range(NBUF)]
            + [
                pltpu.VMEM((N_NODES, NFEAT), jnp.float32),
                pltpu.VMEM((N_NODES, NHID), jnp.float32),
                pltpu.SemaphoreType.DMA,
                pltpu.SemaphoreType.DMA((NBUF,)),
                pltpu.SemaphoreType.DMA((NBUF,)),
                pltpu.SemaphoreType.DMA((NBUF,)),
            ]
        ),
    )(x, adj, W, b2, mu)
    return (out, q)


# speedup vs baseline: 1.0603x; 1.0549x over previous
import jax
import jax.numpy as jnp
from jax.experimental import pallas as pl
from jax.experimental.pallas import tpu as pltpu

NFEAT = 128
NHID = 32
ALPHA = 0.2
N_NODES = 10000
N_CLUSTERS = 10

BR = 400
BH = BR // 2
NI = N_NODES // BR


def _soft_assign(o, mu_ref):
    cols = []
    for c in range(N_CLUSTERS):
        d = o - mu_ref[c:c + 1, :]
        cols.append(jnp.sum(d * d, axis=1, keepdims=True))
    dist2 = jnp.concatenate(cols, axis=1)
    qv = 1.0 / (1.0 + dist2 / ALPHA + 1e-8)
    p = jnp.exp((ALPHA + 1.0) * jnp.log(qv))
    return p / jnp.sum(p, axis=1, keepdims=True)


def _gcdec_kernel(x_ref, adj_top_ref, adj_bot_ref, w_ref, b_ref, mu_ref,
                  out_ref, q_ref, support_ref):
    i = pl.program_id(0)

    @pl.when(i == 0)
    def _():
        support_ref[...] = jnp.dot(x_ref[...], w_ref[...],
                                   preferred_element_type=jnp.float32)

    o_top = jnp.dot(adj_top_ref[...], support_ref[...],
                    preferred_element_type=jnp.float32) + b_ref[...]
    out_ref[:BH, :] = o_top
    q_ref[:BH, :] = _soft_assign(o_top, mu_ref)
    o_bot = jnp.dot(adj_bot_ref[...], support_ref[...],
                    preferred_element_type=jnp.float32) + b_ref[...]
    out_ref[BH:, :] = o_bot
    q_ref[BH:, :] = _soft_assign(o_bot, mu_ref)


@jax.jit
def kernel(x, adj, W, b, mu):
    b2 = b.reshape(1, NHID)
    out, q = pl.pallas_call(
        _gcdec_kernel,
        grid=(NI,),
        in_specs=[
            pl.BlockSpec((N_NODES, NFEAT), lambda i: (0, 0)),
            pl.BlockSpec((BH, N_NODES), lambda i: (2 * i, 0)),
            pl.BlockSpec((BH, N_NODES), lambda i: (2 * i + 1, 0)),
            pl.BlockSpec((NFEAT, NHID), lambda i: (0, 0)),
            pl.BlockSpec((1, NHID), lambda i: (0, 0)),
            pl.BlockSpec((N_CLUSTERS, NHID), lambda i: (0, 0)),
        ],
        out_specs=[
            pl.BlockSpec((BR, NHID), lambda i: (i, 0)),
            pl.BlockSpec((BR, N_CLUSTERS), lambda i: (i, 0)),
        ],
        out_shape=[
            jax.ShapeDtypeStruct((N_NODES, NHID), jnp.float32),
            jax.ShapeDtypeStruct((N_NODES, N_CLUSTERS), jnp.float32),
        ],
        scratch_shapes=[
            pltpu.VMEM((N_NODES, NHID), jnp.float32),
        ],
    )(x, adj, adj, W, b2, mu)
    return (out, q)


# speedup vs baseline: 1.0764x; 1.0152x over previous
import jax
import jax.numpy as jnp
from jax.experimental import pallas as pl
from jax.experimental.pallas import tpu as pltpu

NFEAT = 128
NHID = 32
ALPHA = 0.2
N_NODES = 10000
N_CLUSTERS = 10

BR = 400
NI = N_NODES // BR


def _gcdec_kernel(x_ref, adj_ref, w_ref, b_ref, mu_ref, out_ref, q_ref,
                  support_ref):
    i = pl.program_id(0)

    @pl.when(i == 0)
    def _():
        support = jnp.dot(x_ref[...], w_ref[...],
                          preferred_element_type=jnp.float32)
        support_ref[...] = support.astype(jnp.bfloat16)

    o = jnp.dot(adj_ref[...].astype(jnp.bfloat16), support_ref[...],
                preferred_element_type=jnp.float32) + b_ref[...]
    out_ref[...] = o

    cols = []
    for c in range(N_CLUSTERS):
        d = o - mu_ref[c:c + 1, :]
        cols.append(jnp.sum(d * d, axis=1, keepdims=True))
    dist2 = jnp.concatenate(cols, axis=1)
    qv = 1.0 / (1.0 + dist2 / ALPHA + 1e-8)
    p = jnp.exp((ALPHA + 1.0) * jnp.log(qv))
    q_ref[...] = p / jnp.sum(p, axis=1, keepdims=True)


@jax.jit
def kernel(x, adj, W, b, mu):
    b2 = b.reshape(1, NHID)
    out, q = pl.pallas_call(
        _gcdec_kernel,
        grid=(NI,),
        in_specs=[
            pl.BlockSpec((N_NODES, NFEAT), lambda i: (0, 0)),
            pl.BlockSpec((BR, N_NODES), lambda i: (i, 0)),
            pl.BlockSpec((NFEAT, NHID), lambda i: (0, 0)),
            pl.BlockSpec((1, NHID), lambda i: (0, 0)),
            pl.BlockSpec((N_CLUSTERS, NHID), lambda i: (0, 0)),
        ],
        out_specs=[
            pl.BlockSpec((BR, NHID), lambda i: (i, 0)),
            pl.BlockSpec((BR, N_CLUSTERS), lambda i: (i, 0)),
        ],
        out_shape=[
            jax.ShapeDtypeStruct((N_NODES, NHID), jnp.float32),
            jax.ShapeDtypeStruct((N_NODES, N_CLUSTERS), jnp.float32),
        ],
        scratch_shapes=[
            pltpu.VMEM((N_NODES, NHID), jnp.bfloat16),
        ],
    )(x, adj, W, b2, mu)
    return (out, q)
